# baseline (speedup 1.0000x reference)
import functools

import jax
import jax.numpy as jnp
from jax import lax
from jax.experimental import pallas as pl
from jax.experimental.pallas import tpu as pltpu
from jax.experimental.pallas import tpu_sc as plsc

N_RAW = 10000
E_RAW = 320000
F = 128
N_P = 10240
E_P = 327680
NC, NS = 2, 16
NW = NC * NS
E_TILE = E_P // NW
CHUNK = 128
N_CHUNKS = E_TILE // CHUNK
G_CHUNKS = 8
N_GROUPS = N_CHUNKS // G_CHUNKS
ROWS_TILE = N_P // NS

_mesh = plsc.VectorSubcoreMesh(
    core_axis_name="c", subcore_axis_name="s", num_cores=NC, num_subcores=NS
)


def _zero_fill(ref, n_rows, n_cols):
    zero = jnp.zeros((16,), jnp.float32)

    def body(i, carry):
        for k in range(n_cols // 16):
            ref[i, pl.ds(k * 16, 16)] = zero
        return carry

    lax.fori_loop(0, n_rows, body, 0)


def _deg_body(src_hbm, dst_hbm, out_hbm, idx_s, idx_d, cnt_s, cnt_d,
              red_v, stage):
    c = lax.axis_index("c")
    s = lax.axis_index("s")
    wid = c * NS + s
    pltpu.sync_copy(src_hbm.at[wid], idx_s)
    pltpu.sync_copy(dst_hbm.at[wid], idx_d)
    zero = jnp.zeros((16,), jnp.float32)
    one = jnp.ones((16,), jnp.float32)

    def zboth(i, carry):
        cnt_s[pl.ds(i * 16, 16)] = zero
        cnt_d[pl.ds(i * 16, 16)] = zero
        return carry

    lax.fori_loop(0, N_P // 16, zboth, 0)

    def grp(g, carry):
        for cc in range(G_CHUNKS):
            for k in range(CHUNK // 16):
                iv_s = idx_s[g, cc, pl.ds(k * 16, 16)]
                plsc.addupdate_scatter(cnt_s, [iv_s], one)
                iv_d = idx_d[g, cc, pl.ds(k * 16, 16)]
                plsc.addupdate_scatter(cnt_d, [iv_d], one)
        return carry

    lax.fori_loop(0, N_GROUPS, grp, 0)
    pltpu.sync_copy(cnt_s, stage.at[s, 0])
    pltpu.sync_copy(cnt_d, stage.at[s, 1])
    plsc.subcore_barrier()
    base = s * ROWS_TILE
    for which in range(2):
        for t in range(NS):
            pltpu.sync_copy(stage.at[t, which, pl.ds(base, ROWS_TILE)],
                            red_v.at[t])

        def red(g, carry):
            acc = red_v[0, pl.ds(g * 16, 16)]
            for t in range(1, NS):
                acc = acc + red_v[t, pl.ds(g * 16, 16)]
            out_row = cnt_s if which == 0 else cnt_d
            out_row[pl.ds(g * 16, 16)] = acc
            return carry

        lax.fori_loop(0, ROWS_TILE // 16, red, 0)
        dst_ref = cnt_s if which == 0 else cnt_d
        pltpu.sync_copy(dst_ref.at[pl.ds(0, ROWS_TILE)],
                        out_hbm.at[c, which, pl.ds(base, ROWS_TILE)])


def _prop_body(table_hbm, src_hbm, dst_hbm, out_hbm, idx_s, idx_d,
               rb0, rb1, agg, gs0, gs1, is0, is1):
    c = lax.axis_index("c")
    s = lax.axis_index("s")
    wid = c * NS + s
    rows = (rb0, rb1)
    gsem = (gs0, gs1)
    isem = (is0, is1)
    _zero_fill(rb0, CHUNK, F)
    for r in range(ROWS_TILE // CHUNK):
        base = s * ROWS_TILE + r * CHUNK
        pltpu.sync_copy(rb0, agg.at[pl.ds(base, CHUNK)])
    plsc.subcore_barrier()

    def ia_start(g, slot):
        pltpu.async_copy(src_hbm.at[wid, g], idx_s.at[slot], isem[slot])
        pltpu.async_copy(dst_hbm.at[wid, g], idx_d.at[slot], isem[slot])

    def ia_wait(slot):
        pltpu.make_async_copy(src_hbm.at[0, 0], idx_s.at[slot],
                              isem[slot]).wait()
        pltpu.make_async_copy(dst_hbm.at[0, 0], idx_d.at[slot],
                              isem[slot]).wait()

    def g_start(islot, cc, k):
        pltpu.async_copy(table_hbm.at[idx_s.at[islot, cc]], rows[k], gsem[k])

    def g_wait(k):
        pltpu.make_async_copy(table_hbm.at[pl.ds(0, CHUNK)], rows[k],
                              gsem[k]).wait()

    def group(g, islot, has_next):
        if has_next:
            ia_start(g + 1, 1 - islot)
        for cc in range(G_CHUNKS):
            k = cc % 2
            g_wait(k)
            if cc + 1 < G_CHUNKS:
                g_start(islot, cc + 1, 1 - k)
            elif has_next:
                ia_wait(1 - islot)
                g_start(1 - islot, 0, 1 - k)
            pltpu.sync_copy(rows[k], agg.at[idx_d.at[islot, cc]], add=True)

    ia_start(0, 0)
    ia_wait(0)
    g_start(0, 0, 0)

    def two_groups(j, carry):
        group(2 * j, 0, True)
        group(2 * j + 1, 1, True)
        return carry

    lax.fori_loop(0, N_GROUPS // 2 - 1, two_groups, 0)
    group(N_GROUPS - 2, 0, True)
    group(N_GROUPS - 1, 1, False)
    plsc.subcore_barrier()
    pltpu.sync_copy(agg.at[pl.ds(s * ROWS_TILE, ROWS_TILE)],
                    out_hbm.at[c, pl.ds(s * ROWS_TILE, ROWS_TILE)])


_DEG_SCRATCH = [
    pltpu.VMEM((N_GROUPS, G_CHUNKS, CHUNK), jnp.int32),
    pltpu.VMEM((N_GROUPS, G_CHUNKS, CHUNK), jnp.int32),
    pltpu.VMEM((N_P,), jnp.float32),
    pltpu.VMEM((N_P,), jnp.float32),
    pltpu.VMEM((NS, ROWS_TILE), jnp.float32),
    pltpu.VMEM_SHARED((NS, 2, N_P), jnp.float32),
]
_PROP_SCRATCH = [
    pltpu.VMEM((2, G_CHUNKS, CHUNK), jnp.int32),
    pltpu.VMEM((2, G_CHUNKS, CHUNK), jnp.int32),
    pltpu.VMEM((CHUNK, F), jnp.float32),
    pltpu.VMEM((CHUNK, F), jnp.float32),
    pltpu.VMEM_SHARED((N_P, F), jnp.float32),
] + [pltpu.SemaphoreType.DMA] * 4

_deg_kernel = pl.kernel(
    _deg_body,
    out_type=jax.ShapeDtypeStruct((NC, 2, N_P), jnp.float32),
    mesh=_mesh, scratch_types=_DEG_SCRATCH,
    compiler_params=pltpu.CompilerParams(needs_layout_passes=False))

_prop_kernel = pl.kernel(
    _prop_body,
    out_type=jax.ShapeDtypeStruct((NC, N_P, F), jnp.float32),
    mesh=_mesh, scratch_types=_PROP_SCRATCH)


def _deg_inv(cnt_pair):
    cnt = cnt_pair[0] + cnt_pair[1]
    return lax.rsqrt(jnp.maximum(cnt, 1.0))


def _tc_mid_body(p_ref, cs_ref, cd_ref, flag_ref, b_ref, w_ref, out_ref):
    p = p_ref[0] + p_ref[1]
    f = flag_ref[0, 0]
    ddst = _deg_inv(cd_ref[...])
    scale = jnp.where(f > 0, ddst, jnp.ones_like(ddst))
    h = p * scale + f * b_ref[...]
    h = jnp.where(f > 0, jnp.maximum(h, 0.0), h)
    dsrc = _deg_inv(cs_ref[...])
    out_ref[...] = jnp.dot(h * dsrc, w_ref[...],
                           preferred_element_type=jnp.float32)


def _tc_last_body(p_ref, cd_ref, b_ref, out_ref):
    p = p_ref[0] + p_ref[1]
    ddst = _deg_inv(cd_ref[...])
    out_ref[...] = p * ddst + b_ref[...]


_BLK = 1024
_GRID = N_P // _BLK

_cnt_spec = pl.BlockSpec((2, _BLK, 1), lambda i: (0, i, 0))
_p_spec = pl.BlockSpec((2, _BLK, F), lambda i: (0, i, 0))
_w_spec = pl.BlockSpec((F, F), lambda i: (0, 0))
_b_spec = pl.BlockSpec((1, F), lambda i: (0, 0))
_flag_spec = pl.BlockSpec((1, 1), lambda i: (0, 0))
_out_spec = pl.BlockSpec((_BLK, F), lambda i: (i, 0))
_out_shape = jax.ShapeDtypeStruct((N_P, F), jnp.float32)

_tc_mid = pl.pallas_call(
    _tc_mid_body, grid=(_GRID,),
    in_specs=[_p_spec, _cnt_spec, _cnt_spec, _flag_spec, _b_spec, _w_spec],
    out_specs=_out_spec, out_shape=_out_shape)

_tc_last = pl.pallas_call(
    _tc_last_body, grid=(_GRID,),
    in_specs=[_p_spec, _cnt_spec, _b_spec],
    out_specs=_out_spec, out_shape=_out_shape)


def kernel(features, edge_index, W0, b0, W1, b1, W2, b2):
    feat_p = jnp.pad(features, ((0, N_P - N_RAW), (0, 0)))
    pad_edges = jnp.full((2, E_P - E_RAW), N_RAW, jnp.int32)
    ei = jnp.concatenate([edge_index, pad_edges], axis=1)
    src3 = ei[0].reshape(NW, N_GROUPS, G_CHUNKS, CHUNK)
    dst3 = ei[1].reshape(NW, N_GROUPS, G_CHUNKS, CHUNK)

    cnt = _deg_kernel(src3, dst3)
    cnt_s = cnt[:, 0].reshape(NC, N_P, 1)
    cnt_d = cnt[:, 1].reshape(NC, N_P, 1)
    w_stack = jnp.stack([W0, W1, W2])
    b_stack = jnp.stack([jnp.zeros_like(b0), b0, b1])

    p = jnp.stack([feat_p, jnp.zeros_like(feat_p)])
    for i in range(3):
        flag = jnp.full((1, 1), float(i > 0), jnp.float32)
        t = _tc_mid(p, cnt_s, cnt_d, flag, b_stack[i].reshape(1, F),
                    w_stack[i])
        p = _prop_kernel(t, src3, dst3)
    out = _tc_last(p, cnt_d, b2.reshape(1, F))
    return out[:N_RAW]


# speedup vs baseline: 1.2466x; 1.2466x over previous
import functools

import jax
import jax.numpy as jnp
from jax import lax
from jax.experimental import pallas as pl
from jax.experimental.pallas import tpu as pltpu
from jax.experimental.pallas import tpu_sc as plsc

N_RAW = 10000
E_RAW = 320000
F = 128
N_P = 10240
E_P = 327680
NC, NS = 2, 16
NW = NC * NS
E_TILE = E_P // NW
SUB = 64
G_SUB = 16
N_GROUPS = E_TILE // (G_SUB * SUB)
NRING = 4
ROWS_TILE = N_P // NS

_mesh = plsc.VectorSubcoreMesh(
    core_axis_name="c", subcore_axis_name="s", num_cores=NC, num_subcores=NS
)


def _zero_fill(ref, n_rows, n_cols):
    zero = jnp.zeros((16,), jnp.float32)

    def body(i, carry):
        for k in range(n_cols // 16):
            ref[i, pl.ds(k * 16, 16)] = zero
        return carry

    lax.fori_loop(0, n_rows, body, 0)


def _deg_body(src_hbm, dst_hbm, out_hbm, idx_s, idx_d, cnt_s, cnt_d,
              red_v, stage):
    c = lax.axis_index("c")
    s = lax.axis_index("s")
    wid = c * NS + s
    pltpu.sync_copy(src_hbm.at[wid], idx_s)
    pltpu.sync_copy(dst_hbm.at[wid], idx_d)
    zero = jnp.zeros((16,), jnp.float32)
    one = jnp.ones((16,), jnp.float32)

    def zboth(i, carry):
        cnt_s[pl.ds(i * 16, 16)] = zero
        cnt_d[pl.ds(i * 16, 16)] = zero
        return carry

    lax.fori_loop(0, N_P // 16, zboth, 0)

    def grp(g, carry):
        for cc in range(G_SUB):
            for k in range(SUB // 16):
                iv_s = idx_s[g, cc, pl.ds(k * 16, 16)]
                plsc.addupdate_scatter(cnt_s, [iv_s], one)
                iv_d = idx_d[g, cc, pl.ds(k * 16, 16)]
                plsc.addupdate_scatter(cnt_d, [iv_d], one)
        return carry

    lax.fori_loop(0, N_GROUPS, grp, 0)
    pltpu.sync_copy(cnt_s, stage.at[s, 0])
    pltpu.sync_copy(cnt_d, stage.at[s, 1])
    plsc.subcore_barrier()
    base = s * ROWS_TILE
    for which in range(2):
        for t in range(NS):
            pltpu.sync_copy(stage.at[t, which, pl.ds(base, ROWS_TILE)],
                            red_v.at[t])

        def red(g, carry):
            acc = red_v[0, pl.ds(g * 16, 16)]
            for t in range(1, NS):
                acc = acc + red_v[t, pl.ds(g * 16, 16)]
            out_row = cnt_s if which == 0 else cnt_d
            out_row[pl.ds(g * 16, 16)] = acc
            return carry

        lax.fori_loop(0, ROWS_TILE // 16, red, 0)
        dst_ref = cnt_s if which == 0 else cnt_d
        pltpu.sync_copy(dst_ref.at[pl.ds(0, ROWS_TILE)],
                        out_hbm.at[c, which, pl.ds(base, ROWS_TILE)])


def _prop_body(table_hbm, src_hbm, dst_hbm, out_hbm, idx_s, idx_d,
               rb0, rb1, rb2, rb3, agg, gs0, gs1, gs2, gs3, is0, is1):
    c = lax.axis_index("c")
    s = lax.axis_index("s")
    wid = c * NS + s
    rows = (rb0, rb1, rb2, rb3)
    gsem = (gs0, gs1, gs2, gs3)
    isem = (is0, is1)
    _zero_fill(rb0, SUB, F)
    for r in range(ROWS_TILE // SUB):
        base = s * ROWS_TILE + r * SUB
        pltpu.sync_copy(rb0, agg.at[pl.ds(base, SUB)])
    plsc.subcore_barrier()

    def ia_start(g, slot):
        pltpu.async_copy(src_hbm.at[wid, g], idx_s.at[slot], isem[slot])
        pltpu.async_copy(dst_hbm.at[wid, g], idx_d.at[slot], isem[slot])

    def ia_wait(slot):
        pltpu.make_async_copy(src_hbm.at[0, 0], idx_s.at[slot],
                              isem[slot]).wait()
        pltpu.make_async_copy(dst_hbm.at[0, 0], idx_d.at[slot],
                              isem[slot]).wait()

    def g_start(islot, cc, k):
        pltpu.async_copy(table_hbm.at[idx_s.at[islot, cc]], rows[k], gsem[k])

    def g_wait(k):
        pltpu.make_async_copy(table_hbm.at[pl.ds(0, SUB)], rows[k],
                              gsem[k]).wait()

    def group(g, islot, has_next):
        if has_next:
            ia_start(g + 1, 1 - islot)
        for t in range(G_SUB):
            k = t % NRING
            g_wait(k)
            pltpu.sync_copy(rows[k], agg.at[idx_d.at[islot, t]], add=True)
            tp = t + NRING
            if tp < G_SUB:
                g_start(islot, tp, k)
            elif has_next:
                if tp == G_SUB:
                    ia_wait(1 - islot)
                g_start(1 - islot, tp - G_SUB, k)

    ia_start(0, 0)
    ia_wait(0)
    for k in range(NRING):
        g_start(0, k, k)

    def two_groups(j, carry):
        group(2 * j, 0, True)
        group(2 * j + 1, 1, True)
        return carry

    lax.fori_loop(0, N_GROUPS // 2 - 1, two_groups, 0)
    group(N_GROUPS - 2, 0, True)
    group(N_GROUPS - 1, 1, False)
    plsc.subcore_barrier()
    pltpu.sync_copy(agg.at[pl.ds(s * ROWS_TILE, ROWS_TILE)],
                    out_hbm.at[c, pl.ds(s * ROWS_TILE, ROWS_TILE)])


_DEG_SCRATCH = [
    pltpu.VMEM((N_GROUPS, G_SUB, SUB), jnp.int32),
    pltpu.VMEM((N_GROUPS, G_SUB, SUB), jnp.int32),
    pltpu.VMEM((N_P,), jnp.float32),
    pltpu.VMEM((N_P,), jnp.float32),
    pltpu.VMEM((NS, ROWS_TILE), jnp.float32),
    pltpu.VMEM_SHARED((NS, 2, N_P), jnp.float32),
]
_PROP_SCRATCH = [
    pltpu.VMEM((2, G_SUB, SUB), jnp.int32),
    pltpu.VMEM((2, G_SUB, SUB), jnp.int32),
    pltpu.VMEM((SUB, F), jnp.float32),
    pltpu.VMEM((SUB, F), jnp.float32),
    pltpu.VMEM((SUB, F), jnp.float32),
    pltpu.VMEM((SUB, F), jnp.float32),
    pltpu.VMEM_SHARED((N_P, F), jnp.float32),
] + [pltpu.SemaphoreType.DMA] * 6

_deg_kernel = pl.kernel(
    _deg_body,
    out_type=jax.ShapeDtypeStruct((NC, 2, N_P), jnp.float32),
    mesh=_mesh, scratch_types=_DEG_SCRATCH,
    compiler_params=pltpu.CompilerParams(needs_layout_passes=False))

_prop_kernel = pl.kernel(
    _prop_body,
    out_type=jax.ShapeDtypeStruct((NC, N_P, F), jnp.float32),
    mesh=_mesh, scratch_types=_PROP_SCRATCH)


def _deg_inv(cnt_pair):
    cnt = cnt_pair[0] + cnt_pair[1]
    return lax.rsqrt(jnp.maximum(cnt, 1.0))


def _tc_mid_body(p_ref, cs_ref, cd_ref, flag_ref, b_ref, w_ref, out_ref):
    p = p_ref[0] + p_ref[1]
    f = flag_ref[0, 0]
    ddst = _deg_inv(cd_ref[...])
    scale = jnp.where(f > 0, ddst, jnp.ones_like(ddst))
    h = p * scale + f * b_ref[...]
    h = jnp.where(f > 0, jnp.maximum(h, 0.0), h)
    dsrc = _deg_inv(cs_ref[...])
    out_ref[...] = jnp.dot(h * dsrc, w_ref[...],
                           preferred_element_type=jnp.float32)


def _tc_last_body(p_ref, cd_ref, b_ref, out_ref):
    p = p_ref[0] + p_ref[1]
    ddst = _deg_inv(cd_ref[...])
    out_ref[...] = p * ddst + b_ref[...]


_BLK = 1024
_GRID = N_P // _BLK

_cnt_spec = pl.BlockSpec((2, _BLK, 1), lambda i: (0, i, 0))
_p_spec = pl.BlockSpec((2, _BLK, F), lambda i: (0, i, 0))
_w_spec = pl.BlockSpec((F, F), lambda i: (0, 0))
_b_spec = pl.BlockSpec((1, F), lambda i: (0, 0))
_flag_spec = pl.BlockSpec((1, 1), lambda i: (0, 0))
_out_spec = pl.BlockSpec((_BLK, F), lambda i: (i, 0))
_out_shape = jax.ShapeDtypeStruct((N_P, F), jnp.float32)

_tc_mid = pl.pallas_call(
    _tc_mid_body, grid=(_GRID,),
    in_specs=[_p_spec, _cnt_spec, _cnt_spec, _flag_spec, _b_spec, _w_spec],
    out_specs=_out_spec, out_shape=_out_shape)

_tc_last = pl.pallas_call(
    _tc_last_body, grid=(_GRID,),
    in_specs=[_p_spec, _cnt_spec, _b_spec],
    out_specs=_out_spec, out_shape=_out_shape)


def kernel(features, edge_index, W0, b0, W1, b1, W2, b2):
    feat_p = jnp.pad(features, ((0, N_P - N_RAW), (0, 0)))
    pad_edges = jnp.full((2, E_P - E_RAW), N_RAW, jnp.int32)
    ei = jnp.concatenate([edge_index, pad_edges], axis=1)
    src3 = ei[0].reshape(NW, N_GROUPS, G_SUB, SUB)
    dst3 = ei[1].reshape(NW, N_GROUPS, G_SUB, SUB)

    cnt = _deg_kernel(src3, dst3)
    cnt_s = cnt[:, 0].reshape(NC, N_P, 1)
    cnt_d = cnt[:, 1].reshape(NC, N_P, 1)
    w_stack = jnp.stack([W0, W1, W2])
    b_stack = jnp.stack([jnp.zeros_like(b0), b0, b1])

    p = jnp.stack([feat_p, jnp.zeros_like(feat_p)])
    for i in range(3):
        flag = jnp.full((1, 1), float(i > 0), jnp.float32)
        t = _tc_mid(p, cnt_s, cnt_d, flag, b_stack[i].reshape(1, F),
                    w_stack[i])
        p = _prop_kernel(t, src3, dst3)
    out = _tc_last(p, cnt_d, b2.reshape(1, F))
    return out[:N_RAW]


# speedup vs baseline: 1.2478x; 1.0010x over previous
import jax
import jax.numpy as jnp
from jax import lax
from jax.experimental import pallas as pl
from jax.experimental.pallas import tpu as pltpu
from jax.experimental.pallas import tpu_sc as plsc

N_RAW = 10000
E_RAW = 320000
F = 128
N_P = 10240
E_P = 327680
NC, NS = 2, 16
NW = NC * NS
E_TILE = E_P // NW
SUB = 64
G_SUB = 16
N_GROUPS = E_TILE // (G_SUB * SUB)
NRING = 4
ROWS_TILE = N_P // NS

_mesh = plsc.VectorSubcoreMesh(
    core_axis_name="c", subcore_axis_name="s", num_cores=NC, num_subcores=NS
)


def _zero_fill(ref, n_rows, n_cols):
    zero = jnp.zeros((16,), jnp.float32)

    def body(i, carry):
        for k in range(n_cols // 16):
            ref[i, pl.ds(k * 16, 16)] = zero
        return carry

    lax.fori_loop(0, n_rows, body, 0)


def _deg_body(src_hbm, dst_hbm, out_hbm, idx_s, idx_d, cnt_s, cnt_d,
              red_v, stage):
    c = lax.axis_index("c")
    s = lax.axis_index("s")
    wid = c * NS + s
    pltpu.sync_copy(src_hbm.at[wid], idx_s)
    pltpu.sync_copy(dst_hbm.at[wid], idx_d)
    zero = jnp.zeros((16,), jnp.float32)
    one = jnp.ones((16,), jnp.float32)

    def zboth(i, carry):
        cnt_s[pl.ds(i * 16, 16)] = zero
        cnt_d[pl.ds(i * 16, 16)] = zero
        return carry

    lax.fori_loop(0, N_P // 16, zboth, 0)

    def grp(g, carry):
        for cc in range(G_SUB):
            for k in range(SUB // 16):
                iv_s = idx_s[g, cc, pl.ds(k * 16, 16)]
                plsc.addupdate_scatter(cnt_s, [iv_s], one)
                iv_d = idx_d[g, cc, pl.ds(k * 16, 16)]
                plsc.addupdate_scatter(cnt_d, [iv_d], one)
        return carry

    lax.fori_loop(0, N_GROUPS, grp, 0)
    pltpu.sync_copy(cnt_s, stage.at[s, 0])
    pltpu.sync_copy(cnt_d, stage.at[s, 1])
    plsc.subcore_barrier()
    base = s * ROWS_TILE
    for which in range(2):
        for t in range(NS):
            pltpu.sync_copy(stage.at[t, which, pl.ds(base, ROWS_TILE)],
                            red_v.at[t])

        def red(g, carry):
            acc = red_v[0, pl.ds(g * 16, 16)]
            for t in range(1, NS):
                acc = acc + red_v[t, pl.ds(g * 16, 16)]
            out_row = cnt_s if which == 0 else cnt_d
            out_row[pl.ds(g * 16, 16)] = acc
            return carry

        lax.fori_loop(0, ROWS_TILE // 16, red, 0)
        dst_ref = cnt_s if which == 0 else cnt_d
        pltpu.sync_copy(dst_ref.at[pl.ds(0, ROWS_TILE)],
                        out_hbm.at[c, which, pl.ds(base, ROWS_TILE)])


def _prop_body(table_hbm, src_hbm, dst_hbm, out_hbm, idx_s, idx_d,
               rb0, rb1, rb2, rb3, agg, gs0, gs1, gs2, gs3, is0, is1):
    c = lax.axis_index("c")
    s = lax.axis_index("s")
    wid = c * NS + s
    rows = (rb0, rb1, rb2, rb3)
    gsem = (gs0, gs1, gs2, gs3)
    isem = (is0, is1)
    _zero_fill(rb0, SUB, F)
    for r in range(ROWS_TILE // SUB):
        base = s * ROWS_TILE + r * SUB
        pltpu.sync_copy(rb0, agg.at[pl.ds(base, SUB)])
    plsc.subcore_barrier()

    def ia_start(g, slot):
        pltpu.async_copy(src_hbm.at[wid, g], idx_s.at[slot], isem[slot])
        pltpu.async_copy(dst_hbm.at[wid, g], idx_d.at[slot], isem[slot])

    def ia_wait(slot):
        pltpu.make_async_copy(src_hbm.at[0, 0], idx_s.at[slot],
                              isem[slot]).wait()
        pltpu.make_async_copy(dst_hbm.at[0, 0], idx_d.at[slot],
                              isem[slot]).wait()

    def g_start(islot, cc, k):
        pltpu.async_copy(table_hbm.at[idx_s.at[islot, cc]], rows[k], gsem[k])

    def g_wait(k):
        pltpu.make_async_copy(table_hbm.at[pl.ds(0, SUB)], rows[k],
                              gsem[k]).wait()

    def group(g, islot, has_next):
        if has_next:
            ia_start(g + 1, 1 - islot)
        for t in range(G_SUB):
            k = t % NRING
            g_wait(k)
            pltpu.sync_copy(rows[k], agg.at[idx_d.at[islot, t]], add=True)
            tp = t + NRING
            if tp < G_SUB:
                g_start(islot, tp, k)
            elif has_next:
                if tp == G_SUB:
                    ia_wait(1 - islot)
                g_start(1 - islot, tp - G_SUB, k)

    ia_start(0, 0)
    ia_wait(0)
    for k in range(NRING):
        g_start(0, k, k)

    def two_groups(j, carry):
        group(2 * j, 0, True)
        group(2 * j + 1, 1, True)
        return carry

    lax.fori_loop(0, N_GROUPS // 2 - 1, two_groups, 0)
    group(N_GROUPS - 2, 0, True)
    group(N_GROUPS - 1, 1, False)
    plsc.subcore_barrier()
    pltpu.sync_copy(agg.at[pl.ds(s * ROWS_TILE, ROWS_TILE)],
                    out_hbm.at[c, pl.ds(s * ROWS_TILE, ROWS_TILE)])


_DEG_SCRATCH = [
    pltpu.VMEM((N_GROUPS, G_SUB, SUB), jnp.int32),
    pltpu.VMEM((N_GROUPS, G_SUB, SUB), jnp.int32),
    pltpu.VMEM((N_P,), jnp.float32),
    pltpu.VMEM((N_P,), jnp.float32),
    pltpu.VMEM((NS, ROWS_TILE), jnp.float32),
    pltpu.VMEM_SHARED((NS, 2, N_P), jnp.float32),
]
_PROP_SCRATCH = [
    pltpu.VMEM((2, G_SUB, SUB), jnp.int32),
    pltpu.VMEM((2, G_SUB, SUB), jnp.int32),
    pltpu.VMEM((SUB, F), jnp.float32),
    pltpu.VMEM((SUB, F), jnp.float32),
    pltpu.VMEM((SUB, F), jnp.float32),
    pltpu.VMEM((SUB, F), jnp.float32),
    pltpu.VMEM_SHARED((N_P, F), jnp.float32),
] + [pltpu.SemaphoreType.DMA] * 6

_deg_kernel = pl.kernel(
    _deg_body,
    out_type=jax.ShapeDtypeStruct((NC, 2, N_P), jnp.float32),
    mesh=_mesh, scratch_types=_DEG_SCRATCH,
    compiler_params=pltpu.CompilerParams(needs_layout_passes=False))

_prop_kernel = pl.kernel(
    _prop_body,
    out_type=jax.ShapeDtypeStruct((NC, N_P, F), jnp.float32),
    mesh=_mesh, scratch_types=_PROP_SCRATCH)


def _deg_inv(cnt_pair):
    cnt = cnt_pair[0] + cnt_pair[1]
    return lax.rsqrt(jnp.maximum(cnt, 1.0))


def _tc_mid_body(p_ref, cs_ref, cd_ref, flag_ref, b_ref, w_ref, out_ref):
    p = p_ref[0] + p_ref[1]
    f = flag_ref[0, 0]
    ddst = _deg_inv(cd_ref[...])
    scale = jnp.where(f > 0, ddst, jnp.ones_like(ddst))
    h = p * scale + f * b_ref[...]
    h = jnp.where(f > 0, jnp.maximum(h, 0.0), h)
    dsrc = _deg_inv(cs_ref[...])
    out_ref[...] = jnp.dot(h * dsrc, w_ref[...],
                           preferred_element_type=jnp.float32)


def _tc_last_body(p_ref, cd_ref, b_ref, out_ref):
    p = p_ref[0] + p_ref[1]
    ddst = _deg_inv(cd_ref[...])
    out_ref[...] = p * ddst + b_ref[...]


_BLK = 1024
_GRID = N_P // _BLK

_cnt_spec = pl.BlockSpec((2, _BLK, 1), lambda i: (0, i, 0))
_p_spec = pl.BlockSpec((2, _BLK, F), lambda i: (0, i, 0))
_w_spec = pl.BlockSpec((F, F), lambda i: (0, 0))
_b_spec = pl.BlockSpec((1, F), lambda i: (0, 0))
_flag_spec = pl.BlockSpec((1, 1), lambda i: (0, 0))
_out_spec = pl.BlockSpec((_BLK, F), lambda i: (i, 0))
_out_shape = jax.ShapeDtypeStruct((N_P, F), jnp.float32)

_tc_mid = pl.pallas_call(
    _tc_mid_body, grid=(_GRID,),
    in_specs=[_p_spec, _cnt_spec, _cnt_spec, _flag_spec, _b_spec, _w_spec],
    out_specs=_out_spec, out_shape=_out_shape)

_tc_last = pl.pallas_call(
    _tc_last_body, grid=(_GRID,),
    in_specs=[_p_spec, _cnt_spec, _b_spec],
    out_specs=_out_spec, out_shape=_out_shape)


def kernel(features, edge_index, W0, b0, W1, b1, W2, b2):
    feat_p = jnp.pad(features, ((0, N_P - N_RAW), (0, 0)))
    pad_edges = jnp.full((2, E_P - E_RAW), N_RAW, jnp.int32)
    ei = jnp.concatenate([edge_index, pad_edges], axis=1)
    src3 = ei[0].reshape(NW, N_GROUPS, G_SUB, SUB)
    dst3 = ei[1].reshape(NW, N_GROUPS, G_SUB, SUB)

    cnt = _deg_kernel(src3, dst3)
    cnt_s = cnt[:, 0].reshape(NC, N_P, 1)
    cnt_d = cnt[:, 1].reshape(NC, N_P, 1)
    w_stack = jnp.stack([W0, W1, W2])
    b_stack = jnp.stack([jnp.zeros_like(b0), b0, b1])

    p = jnp.stack([feat_p, jnp.zeros_like(feat_p)])
    for i in range(3):
        flag = jnp.full((1, 1), float(i > 0), jnp.float32)
        t = _tc_mid(p, cnt_s, cnt_d, flag, b_stack[i].reshape(1, F),
                    w_stack[i])
        p = _prop_kernel(t, src3, dst3)
    out = _tc_last(p, cnt_d, b2.reshape(1, F))
    return out[:N_RAW]
